# baseline (speedup 1.0000x reference)
import functools

import jax
import jax.numpy as jnp
from jax import lax
from jax.experimental import pallas as pl
from jax.experimental.pallas import tpu as pltpu
from jax.experimental.pallas import tpu_sc as plsc


def kernel(x, W):
    B, H = x.shape
    V, D = W.shape
    N = B * H
    R = 2
    C = R * H

    mesh = plsc.VectorSubcoreMesh(core_axis_name="c", subcore_axis_name="s")

    @functools.partial(
        pl.kernel,
        mesh=mesh,
        compiler_params=pltpu.CompilerParams(
            use_tc_tiling_on_sc=True, needs_layout_passes=False
        ),
        out_type=jax.ShapeDtypeStruct((B, H, D), jnp.float32),
        scratch_types=[
            pltpu.VMEM((C,), jnp.int32),
            pltpu.VMEM((C,), jnp.int32),
            pltpu.VMEM((C, 128), jnp.float32),
            pltpu.VMEM((C, D), jnp.float32),
            pltpu.SemaphoreType.DMA,
        ],
    )
    def gk(table_hbm, idx_hbm, out_hbm, idx_v, lidx_v, lines_v, rows_v, sem):
        wid = lax.axis_index("s") * 2 + lax.axis_index("c")

        base_row = wid * (B // 32)

        def body(i, carry):
            b0 = base_row + i * R
            pltpu.sync_copy(idx_hbm.at[pl.ds(b0 * H, C)], idx_v)

            def lidx_body(g, c2):
                v = idx_v[pl.ds(g * 16, 16)]
                lidx_v[pl.ds(g * 16, 16)] = lax.shift_right_logical(v, 2)
                return c2

            lax.fori_loop(0, C // 16, lidx_body, 0)
            pltpu.async_copy(table_hbm.at[lidx_v], lines_v, sem).wait()

            def extract_body(g, c2):
                j16 = lax.iota(jnp.int32, 16) + g * 16
                idxv = idx_v[pl.ds(g * 16, 16)]
                col0 = (idxv & 3) * D
                for c in range(D):
                    val = plsc.load_gather(lines_v, [j16, col0 + c])
                    plsc.store_scatter(
                        rows_v, [j16, jnp.full((16,), c, jnp.int32)], val
                    )
                return c2

            lax.fori_loop(0, C // 16, extract_body, 0)
            for r in range(R):
                pltpu.sync_copy(
                    rows_v.at[pl.ds(r * H, H)], out_hbm.at[b0 + r]
                )
            return carry

        lax.fori_loop(0, (B // 32) // R, body, 0)

    W_lines = W.reshape(V // 4, 4 * D)
    return gk(W_lines, x.reshape(N))


# speedup vs baseline: 2.0607x; 2.0607x over previous
import functools

import jax
import jax.numpy as jnp
from jax import lax
from jax.experimental import pallas as pl
from jax.experimental.pallas import tpu as pltpu
from jax.experimental.pallas import tpu_sc as plsc


def kernel(x, W):
    B, H = x.shape
    V, D = W.shape
    N = B * H

    info = plsc.get_sparse_core_info()
    NC, NS = info.num_cores, info.num_subcores
    NW = NC * NS
    rows_per_w = B // NW
    R = 8
    n_chunks = rows_per_w // R
    C = R * H

    mesh = plsc.VectorSubcoreMesh(core_axis_name="c", subcore_axis_name="s")

    @functools.partial(
        pl.kernel,
        mesh=mesh,
        compiler_params=pltpu.CompilerParams(
            use_tc_tiling_on_sc=False, needs_layout_passes=False
        ),
        out_type=jax.ShapeDtypeStruct((B, H, D), jnp.float32),
        scratch_types=[
            pltpu.VMEM((C,), jnp.int32),
            pltpu.VMEM((C, D), jnp.float32),
            pltpu.SemaphoreType.DMA,
            pltpu.SemaphoreType.DMA,
        ],
    )
    def gather_kernel(table_hbm, idx_hbm, out_hbm, idx_v, rows_v, gsem, osem):
        wid = lax.axis_index("s") * NC + lax.axis_index("c")
        base = wid * rows_per_w

        def body(i, carry):
            b0 = base + i * R
            pltpu.sync_copy(idx_hbm.at[pl.ds(b0 * H, C)], idx_v)
            pltpu.async_copy(table_hbm.at[idx_v], rows_v, gsem).wait()
            copies = [
                pltpu.async_copy(
                    rows_v.at[pl.ds(r * H, H)], out_hbm.at[b0 + r], osem
                )
                for r in range(R)
            ]
            for c in copies:
                c.wait()
            return carry

        lax.fori_loop(0, n_chunks, body, 0)

    return gather_kernel(W, x.reshape(N))


# speedup vs baseline: 2.0846x; 1.0116x over previous
import functools

import jax
import jax.numpy as jnp
from jax import lax
from jax.experimental import pallas as pl
from jax.experimental.pallas import tpu as pltpu
from jax.experimental.pallas import tpu_sc as plsc


def kernel(x, W):
    B, H = x.shape
    V, D = W.shape
    N = B * H

    info = plsc.get_sparse_core_info()
    NC, NS = info.num_cores, info.num_subcores
    NW = NC * NS
    rows_per_w = B // NW
    R = 16
    n_chunks = rows_per_w // R
    C = R * H

    mesh = plsc.VectorSubcoreMesh(core_axis_name="c", subcore_axis_name="s")

    @functools.partial(
        pl.kernel,
        mesh=mesh,
        compiler_params=pltpu.CompilerParams(
            use_tc_tiling_on_sc=False, needs_layout_passes=False
        ),
        out_type=jax.ShapeDtypeStruct((B, H, D), jnp.float32),
        scratch_types=[
            pltpu.VMEM((C,), jnp.int32),
            pltpu.VMEM((C, D), jnp.float32),
            pltpu.SemaphoreType.DMA,
            pltpu.SemaphoreType.DMA,
        ],
    )
    def gather_kernel(table_hbm, idx_hbm, out_hbm, idx_v, rows_v, gsem, osem):
        wid = lax.axis_index("s") * NC + lax.axis_index("c")
        base = wid * rows_per_w

        def body(i, carry):
            b0 = base + i * R
            pltpu.sync_copy(idx_hbm.at[pl.ds(b0 * H, C)], idx_v)
            pltpu.async_copy(table_hbm.at[idx_v], rows_v, gsem).wait()
            copies = [
                pltpu.async_copy(
                    rows_v.at[pl.ds(r * H, H)], out_hbm.at[b0 + r], osem
                )
                for r in range(R)
            ]
            for c in copies:
                c.wait()
            return carry

        lax.fori_loop(0, n_chunks, body, 0)

    return gather_kernel(W, x.reshape(N))
